# baseline (speedup 1.0000x reference)
import functools

import jax
import jax.numpy as jnp
from jax import lax
from jax.experimental import pallas as pl
from jax.experimental.pallas import tpu as pltpu
from jax.experimental.pallas import tpu_sc as plsc

_NUM_SAMPLED = 512
_EXTRA = 1.0
_L = 16
_GRP = 8


def _sc_pool(pts_t, bparams, ftab, *, B, N, M, C):
    NC, NS = 2, 16
    NW = NC * NS
    BOXES = B * M
    BPW = BOXES // NW
    NP = N + 8
    K = _NUM_SAMPLED
    GCH = 128
    NCH = K // GCH

    mesh = plsc.VectorSubcoreMesh(
        core_axis_name="c", subcore_axis_name="s",
        num_cores=NC, num_subcores=NS)

    @functools.partial(
        pl.kernel,
        out_type=(
            jax.ShapeDtypeStruct((BOXES * K, C), jnp.float32),
            jax.ShapeDtypeStruct((BOXES * K,), jnp.float32),
            jax.ShapeDtypeStruct((BOXES * K,), jnp.float32),
            jax.ShapeDtypeStruct((BOXES * K,), jnp.float32),
            jax.ShapeDtypeStruct((BOXES,), jnp.int32),
            jax.ShapeDtypeStruct((BOXES * K,), jnp.int32),
        ),
        mesh=mesh,
        compiler_params=pltpu.CompilerParams(needs_layout_passes=False),
        scratch_types=[
            pltpu.VMEM((N,), jnp.float32),
            pltpu.VMEM((N,), jnp.float32),
            pltpu.VMEM((N,), jnp.float32),
            pltpu.VMEM((BPW, _L), jnp.float32),
            pltpu.VMEM((_GRP, K + _L), jnp.int32),
            pltpu.VMEM((NCH, GCH), jnp.int32),
            pltpu.VMEM((K,), jnp.int32),
            pltpu.VMEM((NCH, GCH, C), jnp.float32),
            pltpu.VMEM((K,), jnp.float32),
            pltpu.VMEM((K,), jnp.float32),
            pltpu.VMEM((K,), jnp.float32),
            pltpu.VMEM((BPW,), jnp.int32),
            pltpu.SemaphoreType.DMA,
            pltpu.SemaphoreType.DMA,
            pltpu.SemaphoreType.DMA,
            pltpu.SemaphoreType.DMA,
            pltpu.SemaphoreType.DMA,
            pltpu.SemaphoreType.DMA,
            pltpu.SemaphoreType.DMA,
            pltpu.SemaphoreType.DMA,
        ],
    )
    def pool_kernel(pts_hbm, bp_hbm, ftab_hbm,
                    feat_hbm, x_hbm, y_hbm, z_hbm, flag_hbm, idx_hbm,
                    xs, ys, zs, bp, bufs, gidx, oidx, fbuf,
                    xb, yb, zb, flags, gs0, gs1, gs2, gs3,
                    os0, os1, os2, os3):
        wid = lax.axis_index("s") * NC + lax.axis_index("c")
        base_box = wid * BPW
        batch = base_box // M
        pltpu.sync_copy(pts_hbm.at[batch * 3 + 0], xs)
        pltpu.sync_copy(pts_hbm.at[batch * 3 + 1], ys)
        pltpu.sync_copy(pts_hbm.at[batch * 3 + 2], zs)
        pltpu.sync_copy(bp_hbm.at[pl.ds(base_box, BPW)], bp)
        boff = batch * NP
        zrow = boff + N
        iota = lax.iota(jnp.int32, _L)
        gsems = [gs0, gs1, gs2, gs3]
        osems = [os0, os1, os2, os3]
        flags_vec = jnp.zeros((_L,), jnp.int32)

        for g in range(BPW // _GRP):
            prm = []
            for t in range(_GRP):
                pv = bp[g * _GRP + t]
                prm.append((pv[0], pv[1], pv[2], pv[3], pv[4], pv[5],
                            pv[6], pv[7]))

            def step(i, cs, prm=prm):
                off = i * _L
                px = xs[pl.ds(off, _L)]
                py = ys[pl.ds(off, _L)]
                pz = zs[pl.ds(off, _L)]
                ivec = off + iota
                ncs = []
                for t, (cx, cy, cz, hx, hy, hz, ca, sa) in enumerate(prm):
                    cnt = cs[t]
                    sx = px - cx
                    sy = py - cy
                    lx = sx * ca - sy * sa
                    ly = sx * sa + sy * ca
                    m = ((jnp.abs(pz - cz) <= hz)
                         & (lx > -hx) & (lx < hx)
                         & (ly > -hy) & (ly < hy))
                    mi = jnp.where(m, jnp.int32(1), jnp.int32(0))
                    incl = plsc.cumsum(mi)
                    mm = m & lax.broadcast(cnt < K, (_L,))
                    plsc.store_scatter(
                        bufs, [lax.broadcast(jnp.int32(t), (_L,)),
                               cnt + incl - 1],
                        ivec, mask=mm)
                    ncs.append(cnt + incl[_L - 1])
                return tuple(ncs)

            cs = lax.fori_loop(0, N // _L, step,
                               tuple(jnp.int32(0) for _ in range(_GRP)))

            for t in range(_GRP):
                bj = g * _GRP + t
                cnt = cs[t]
                nonempty = cnt > 0
                safe = lax.broadcast(jnp.maximum(cnt, 1), (_L,))
                fzero = jnp.float32(0.0)
                tb = lax.broadcast(jnp.int32(t), (_L,))
                cpl = GCH // _L

                def samp(c, carry, tb=tb, safe=safe, nonempty=nonempty,
                         fzero=fzero):
                    kv = iota + c * _L
                    p = lax.rem(kv, safe)
                    gi = plsc.load_gather(bufs, [tb, p])
                    gsafe = jnp.where(nonempty, gi, 0)
                    oidx[pl.ds(c * _L, _L)] = gsafe
                    row = lax.broadcast(c // cpl, (_L,))
                    col = lax.rem(c, cpl) * _L + iota
                    plsc.store_scatter(
                        gidx, [row, col],
                        jnp.where(nonempty, gi + boff, zrow))
                    xb[pl.ds(c * _L, _L)] = jnp.where(
                        nonempty, plsc.load_gather(xs, [gsafe]), fzero)
                    yb[pl.ds(c * _L, _L)] = jnp.where(
                        nonempty, plsc.load_gather(ys, [gsafe]), fzero)
                    zb[pl.ds(c * _L, _L)] = jnp.where(
                        nonempty, plsc.load_gather(zs, [gsafe]), fzero)
                    return carry

                lax.fori_loop(0, K // _L, samp, jnp.int32(0))

                row0 = (base_box + bj) * K
                gcps = []
                for r in range(NCH):
                    if bj > 0:
                        pltpu.make_async_copy(
                            fbuf.at[r],
                            feat_hbm.at[pl.ds(0, GCH)], osems[r]).wait()
                    gcps.append(pltpu.async_copy(
                        ftab_hbm.at[gidx.at[r]], fbuf.at[r], gsems[r]))
                for r in range(NCH):
                    gcps[r].wait()
                    pltpu.async_copy(
                        fbuf.at[r],
                        feat_hbm.at[pl.ds(row0 + r * GCH, GCH)], osems[r])
                pltpu.sync_copy(oidx, idx_hbm.at[pl.ds(row0, K)])
                pltpu.sync_copy(xb, x_hbm.at[pl.ds(row0, K)])
                pltpu.sync_copy(yb, y_hbm.at[pl.ds(row0, K)])
                pltpu.sync_copy(zb, z_hbm.at[pl.ds(row0, K)])

                empty = jnp.where(cnt == 0, jnp.int32(1), jnp.int32(0))
                flags_vec = jnp.where(iota == bj, empty, flags_vec)

        for r in range(NCH):
            pltpu.make_async_copy(
                fbuf.at[r], feat_hbm.at[pl.ds(0, GCH)], osems[r]).wait()

        flags[...] = flags_vec
        pltpu.sync_copy(flags, flag_hbm.at[pl.ds(base_box, BPW)])

    return pool_kernel(pts_t, bparams, ftab)


def kernel(points, point_features, boxes3d):
    B, N, _ = points.shape
    M = boxes3d.shape[1]
    C = point_features.shape[2]
    K = _NUM_SAMPLED

    pts_t = jnp.transpose(points, (0, 2, 1)).reshape(B * 3, N)
    rz = boxes3d[..., 6]
    half = (boxes3d[..., 3:6] + 2.0 * _EXTRA) / 2.0
    zcol = jnp.zeros_like(rz)
    bparams = jnp.stack(
        [boxes3d[..., 0], boxes3d[..., 1], boxes3d[..., 2],
         half[..., 0], half[..., 1], half[..., 2],
         jnp.cos(-rz), jnp.sin(-rz)] + [zcol] * (_L - 8),
        axis=-1).reshape(B * M, _L)
    ftab = jnp.concatenate(
        [point_features, jnp.zeros((B, 8, C), jnp.float32)], axis=1
    ).reshape(B * (N + 8), C)

    feat, x, y, z, flags, idx = _sc_pool(
        pts_t, bparams, ftab, B=B, N=N, M=M, C=C)

    xyz = jnp.stack([x, y, z], axis=-1).reshape(B, M, K, 3)
    pooled = jnp.concatenate([xyz, feat.reshape(B, M, K, C)], axis=-1)
    return (pooled, flags.reshape(B, M), idx.reshape(B, M, K))


# speedup vs baseline: 1.1229x; 1.1229x over previous
import functools

import jax
import jax.numpy as jnp
from jax import lax
from jax.experimental import pallas as pl
from jax.experimental.pallas import tpu as pltpu
from jax.experimental.pallas import tpu_sc as plsc

_NUM_SAMPLED = 512
_EXTRA = 1.0
_L = 16
_GRP = 4


def _sc_pool(pts_t, bparams, ftab, *, B, N, M, C):
    NC, NS = 2, 16
    NW = NC * NS
    BOXES = B * M
    BPW = BOXES // NW
    NP = N + 8
    K = _NUM_SAMPLED
    GCH = 128
    NCH = K // GCH

    mesh = plsc.VectorSubcoreMesh(
        core_axis_name="c", subcore_axis_name="s",
        num_cores=NC, num_subcores=NS)

    @functools.partial(
        pl.kernel,
        out_type=(
            jax.ShapeDtypeStruct((BOXES * K, C), jnp.float32),
            jax.ShapeDtypeStruct((BOXES * K,), jnp.float32),
            jax.ShapeDtypeStruct((BOXES * K,), jnp.float32),
            jax.ShapeDtypeStruct((BOXES * K,), jnp.float32),
            jax.ShapeDtypeStruct((BOXES,), jnp.int32),
            jax.ShapeDtypeStruct((BOXES * K,), jnp.int32),
        ),
        mesh=mesh,
        compiler_params=pltpu.CompilerParams(needs_layout_passes=False),
        scratch_types=[
            pltpu.VMEM((N,), jnp.float32),
            pltpu.VMEM((N,), jnp.float32),
            pltpu.VMEM((N,), jnp.float32),
            pltpu.VMEM((BPW, _L), jnp.float32),
            pltpu.VMEM((_GRP, K + _L), jnp.int32),
            pltpu.VMEM((NCH, GCH), jnp.int32),
            pltpu.VMEM((K,), jnp.int32),
            pltpu.VMEM((NCH, GCH, C), jnp.float32),
            pltpu.VMEM((K,), jnp.float32),
            pltpu.VMEM((K,), jnp.float32),
            pltpu.VMEM((K,), jnp.float32),
            pltpu.VMEM((BPW,), jnp.int32),
            pltpu.SemaphoreType.DMA,
            pltpu.SemaphoreType.DMA,
            pltpu.SemaphoreType.DMA,
            pltpu.SemaphoreType.DMA,
            pltpu.SemaphoreType.DMA,
            pltpu.SemaphoreType.DMA,
            pltpu.SemaphoreType.DMA,
            pltpu.SemaphoreType.DMA,
        ],
    )
    def pool_kernel(pts_hbm, bp_hbm, ftab_hbm,
                    feat_hbm, x_hbm, y_hbm, z_hbm, flag_hbm, idx_hbm,
                    xs, ys, zs, bp, bufs, gidx, oidx, fbuf,
                    xb, yb, zb, flags, gs0, gs1, gs2, gs3,
                    os0, os1, os2, os3):
        wid = lax.axis_index("s") * NC + lax.axis_index("c")
        base_box = wid * BPW
        batch = base_box // M
        pltpu.sync_copy(pts_hbm.at[batch * 3 + 0], xs)
        pltpu.sync_copy(pts_hbm.at[batch * 3 + 1], ys)
        pltpu.sync_copy(pts_hbm.at[batch * 3 + 2], zs)
        pltpu.sync_copy(bp_hbm.at[pl.ds(base_box, BPW)], bp)
        boff = batch * NP
        zrow = boff + N
        iota = lax.iota(jnp.int32, _L)
        gsems = [gs0, gs1, gs2, gs3]
        osems = [os0, os1, os2, os3]
        flags_vec = jnp.zeros((_L,), jnp.int32)

        for g in range(BPW // _GRP):
            prm = []
            for t in range(_GRP):
                pv = bp[g * _GRP + t]
                prm.append((pv[0], pv[1], pv[2], pv[3], pv[4], pv[5],
                            pv[6], pv[7]))

            def step(i, cs, prm=prm):
                off = i * _L
                px = xs[pl.ds(off, _L)]
                py = ys[pl.ds(off, _L)]
                pz = zs[pl.ds(off, _L)]
                ivec = off + iota
                ncs = []
                for t, (cx, cy, cz, hx, hy, hz, ca, sa) in enumerate(prm):
                    cnt = cs[t]
                    sx = px - cx
                    sy = py - cy
                    lx = sx * ca - sy * sa
                    ly = sx * sa + sy * ca
                    m = ((jnp.abs(pz - cz) <= hz)
                         & (lx > -hx) & (lx < hx)
                         & (ly > -hy) & (ly < hy))
                    mi = jnp.where(m, jnp.int32(1), jnp.int32(0))
                    incl = plsc.cumsum(mi)
                    mm = m & lax.broadcast(cnt < K, (_L,))
                    plsc.store_scatter(
                        bufs, [lax.broadcast(jnp.int32(t), (_L,)),
                               cnt + incl - 1],
                        ivec, mask=mm)
                    ncs.append(cnt + incl[_L - 1])
                return tuple(ncs)

            cs = lax.fori_loop(0, N // _L, step,
                               tuple(jnp.int32(0) for _ in range(_GRP)),
                               unroll=2)

            for t in range(_GRP):
                bj = g * _GRP + t
                cnt = cs[t]
                nonempty = cnt > 0
                safe = lax.broadcast(jnp.maximum(cnt, 1), (_L,))
                fzero = jnp.float32(0.0)
                tb = lax.broadcast(jnp.int32(t), (_L,))
                cpl = GCH // _L

                def samp(c, carry, tb=tb, safe=safe, nonempty=nonempty,
                         fzero=fzero):
                    kv = iota + c * _L
                    p = lax.rem(kv, safe)
                    gi = plsc.load_gather(bufs, [tb, p])
                    gsafe = jnp.where(nonempty, gi, 0)
                    oidx[pl.ds(c * _L, _L)] = gsafe
                    row = lax.broadcast(c // cpl, (_L,))
                    col = lax.rem(c, cpl) * _L + iota
                    plsc.store_scatter(
                        gidx, [row, col],
                        jnp.where(nonempty, gi + boff, zrow))
                    xb[pl.ds(c * _L, _L)] = jnp.where(
                        nonempty, plsc.load_gather(xs, [gsafe]), fzero)
                    yb[pl.ds(c * _L, _L)] = jnp.where(
                        nonempty, plsc.load_gather(ys, [gsafe]), fzero)
                    zb[pl.ds(c * _L, _L)] = jnp.where(
                        nonempty, plsc.load_gather(zs, [gsafe]), fzero)
                    return carry

                lax.fori_loop(0, K // _L, samp, jnp.int32(0))

                row0 = (base_box + bj) * K
                gcps = []
                for r in range(NCH):
                    if bj > 0:
                        pltpu.make_async_copy(
                            fbuf.at[r],
                            feat_hbm.at[pl.ds(0, GCH)], osems[r]).wait()
                    gcps.append(pltpu.async_copy(
                        ftab_hbm.at[gidx.at[r]], fbuf.at[r], gsems[r]))
                for r in range(NCH):
                    gcps[r].wait()
                    pltpu.async_copy(
                        fbuf.at[r],
                        feat_hbm.at[pl.ds(row0 + r * GCH, GCH)], osems[r])
                pltpu.sync_copy(oidx, idx_hbm.at[pl.ds(row0, K)])
                pltpu.sync_copy(xb, x_hbm.at[pl.ds(row0, K)])
                pltpu.sync_copy(yb, y_hbm.at[pl.ds(row0, K)])
                pltpu.sync_copy(zb, z_hbm.at[pl.ds(row0, K)])

                empty = jnp.where(cnt == 0, jnp.int32(1), jnp.int32(0))
                flags_vec = jnp.where(iota == bj, empty, flags_vec)

        for r in range(NCH):
            pltpu.make_async_copy(
                fbuf.at[r], feat_hbm.at[pl.ds(0, GCH)], osems[r]).wait()

        flags[...] = flags_vec
        pltpu.sync_copy(flags, flag_hbm.at[pl.ds(base_box, BPW)])

    return pool_kernel(pts_t, bparams, ftab)


def kernel(points, point_features, boxes3d):
    B, N, _ = points.shape
    M = boxes3d.shape[1]
    C = point_features.shape[2]
    K = _NUM_SAMPLED

    pts_t = jnp.transpose(points, (0, 2, 1)).reshape(B * 3, N)
    rz = boxes3d[..., 6]
    half = (boxes3d[..., 3:6] + 2.0 * _EXTRA) / 2.0
    zcol = jnp.zeros_like(rz)
    bparams = jnp.stack(
        [boxes3d[..., 0], boxes3d[..., 1], boxes3d[..., 2],
         half[..., 0], half[..., 1], half[..., 2],
         jnp.cos(-rz), jnp.sin(-rz)] + [zcol] * (_L - 8),
        axis=-1).reshape(B * M, _L)
    ftab = jnp.concatenate(
        [point_features, jnp.zeros((B, 8, C), jnp.float32)], axis=1
    ).reshape(B * (N + 8), C)

    feat, x, y, z, flags, idx = _sc_pool(
        pts_t, bparams, ftab, B=B, N=N, M=M, C=C)

    xyz = jnp.stack([x, y, z], axis=-1).reshape(B, M, K, 3)
    pooled = jnp.concatenate([xyz, feat.reshape(B, M, K, C)], axis=-1)
    return (pooled, flags.reshape(B, M), idx.reshape(B, M, K))


# speedup vs baseline: 1.1278x; 1.0044x over previous
import functools

import jax
import jax.numpy as jnp
from jax import lax
from jax.experimental import pallas as pl
from jax.experimental.pallas import tpu as pltpu
from jax.experimental.pallas import tpu_sc as plsc

_NUM_SAMPLED = 512
_EXTRA = 1.0
_L = 16
_GRP = 4


def _sc_pool(pts_t, bparams, ftab, *, B, N, M, C):
    NC, NS = 2, 16
    NW = NC * NS
    BOXES = B * M
    BPW = BOXES // NW
    NP = N + 8
    K = _NUM_SAMPLED
    GCH = 128
    NCH = K // GCH

    mesh = plsc.VectorSubcoreMesh(
        core_axis_name="c", subcore_axis_name="s",
        num_cores=NC, num_subcores=NS)

    @functools.partial(
        pl.kernel,
        out_type=(
            jax.ShapeDtypeStruct((BOXES * K, C), jnp.float32),
            jax.ShapeDtypeStruct((BOXES * K,), jnp.float32),
            jax.ShapeDtypeStruct((BOXES * K,), jnp.float32),
            jax.ShapeDtypeStruct((BOXES * K,), jnp.float32),
            jax.ShapeDtypeStruct((BOXES,), jnp.int32),
            jax.ShapeDtypeStruct((BOXES * K,), jnp.int32),
        ),
        mesh=mesh,
        compiler_params=pltpu.CompilerParams(needs_layout_passes=False),
        scratch_types=[
            pltpu.VMEM((N,), jnp.float32),
            pltpu.VMEM((N,), jnp.float32),
            pltpu.VMEM((N,), jnp.float32),
            pltpu.VMEM((BPW, _L), jnp.float32),
            pltpu.VMEM((_GRP, K + _L), jnp.int32),
            pltpu.VMEM((NCH, GCH), jnp.int32),
            pltpu.VMEM((K,), jnp.int32),
            pltpu.VMEM((NCH, GCH, C), jnp.float32),
            pltpu.VMEM((K,), jnp.float32),
            pltpu.VMEM((K,), jnp.float32),
            pltpu.VMEM((K,), jnp.float32),
            pltpu.VMEM((BPW,), jnp.int32),
            pltpu.SemaphoreType.DMA,
            pltpu.SemaphoreType.DMA,
            pltpu.SemaphoreType.DMA,
            pltpu.SemaphoreType.DMA,
            pltpu.SemaphoreType.DMA,
            pltpu.SemaphoreType.DMA,
            pltpu.SemaphoreType.DMA,
            pltpu.SemaphoreType.DMA,
        ],
    )
    def pool_kernel(pts_hbm, bp_hbm, ftab_hbm,
                    feat_hbm, x_hbm, y_hbm, z_hbm, flag_hbm, idx_hbm,
                    xs, ys, zs, bp, bufs, gidx, oidx, fbuf,
                    xb, yb, zb, flags, gs0, gs1, gs2, gs3,
                    os0, os1, os2, os3):
        wid = lax.axis_index("s") * NC + lax.axis_index("c")
        base_box = wid * BPW
        batch = base_box // M
        pltpu.sync_copy(pts_hbm.at[batch * 3 + 0], xs)
        pltpu.sync_copy(pts_hbm.at[batch * 3 + 1], ys)
        pltpu.sync_copy(pts_hbm.at[batch * 3 + 2], zs)
        pltpu.sync_copy(bp_hbm.at[pl.ds(base_box, BPW)], bp)
        boff = batch * NP
        zrow = boff + N
        iota = lax.iota(jnp.int32, _L)
        gsems = [gs0, gs1, gs2, gs3]
        osems = [os0, os1, os2, os3]
        flags_vec = jnp.zeros((_L,), jnp.int32)

        for g in range(BPW // _GRP):
            prm = []
            for t in range(_GRP):
                pv = bp[g * _GRP + t]
                prm.append((pv[0], pv[1], pv[2], pv[3], pv[4], pv[5],
                            pv[6], pv[7]))

            def step(i, cs, prm=prm):
                off = i * _L
                px = xs[pl.ds(off, _L)]
                py = ys[pl.ds(off, _L)]
                pz = zs[pl.ds(off, _L)]
                ivec = off + iota
                ncs = []
                for t, (cx, cy, cz, hx, hy, hz, ca, sa) in enumerate(prm):
                    cnt = cs[t]
                    sx = px - cx
                    sy = py - cy
                    lx = sx * ca - sy * sa
                    ly = sx * sa + sy * ca
                    m = ((jnp.abs(pz - cz) <= hz)
                         & (lx > -hx) & (lx < hx)
                         & (ly > -hy) & (ly < hy))
                    mi = jnp.where(m, jnp.int32(1), jnp.int32(0))
                    incl = plsc.cumsum(mi)
                    mm = m & lax.broadcast(cnt < K, (_L,))
                    plsc.store_scatter(
                        bufs, [lax.broadcast(jnp.int32(t), (_L,)),
                               cnt + incl - 1],
                        ivec, mask=mm)
                    ncs.append(cnt + incl[_L - 1])
                return tuple(ncs)

            cs = lax.fori_loop(0, N // _L, step,
                               tuple(jnp.int32(0) for _ in range(_GRP)))

            for t in range(_GRP):
                bj = g * _GRP + t
                cnt = cs[t]
                nonempty = cnt > 0
                safe = lax.broadcast(jnp.maximum(cnt, 1), (_L,))
                fzero = jnp.float32(0.0)
                tb = lax.broadcast(jnp.int32(t), (_L,))
                cpl = GCH // _L

                def samp(c, carry, tb=tb, safe=safe, nonempty=nonempty,
                         fzero=fzero):
                    kv = iota + c * _L
                    p = lax.rem(kv, safe)
                    gi = plsc.load_gather(bufs, [tb, p])
                    gsafe = jnp.where(nonempty, gi, 0)
                    oidx[pl.ds(c * _L, _L)] = gsafe
                    row = lax.broadcast(c // cpl, (_L,))
                    col = lax.rem(c, cpl) * _L + iota
                    plsc.store_scatter(
                        gidx, [row, col],
                        jnp.where(nonempty, gi + boff, zrow))
                    xb[pl.ds(c * _L, _L)] = jnp.where(
                        nonempty, plsc.load_gather(xs, [gsafe]), fzero)
                    yb[pl.ds(c * _L, _L)] = jnp.where(
                        nonempty, plsc.load_gather(ys, [gsafe]), fzero)
                    zb[pl.ds(c * _L, _L)] = jnp.where(
                        nonempty, plsc.load_gather(zs, [gsafe]), fzero)
                    return carry

                lax.fori_loop(0, K // _L, samp, jnp.int32(0))

                row0 = (base_box + bj) * K
                gcps = []
                for r in range(NCH):
                    if bj > 0:
                        pltpu.make_async_copy(
                            fbuf.at[r],
                            feat_hbm.at[pl.ds(0, GCH)], osems[r]).wait()
                    gcps.append(pltpu.async_copy(
                        ftab_hbm.at[gidx.at[r]], fbuf.at[r], gsems[r]))
                for r in range(NCH):
                    gcps[r].wait()
                    pltpu.async_copy(
                        fbuf.at[r],
                        feat_hbm.at[pl.ds(row0 + r * GCH, GCH)], osems[r])
                pltpu.sync_copy(oidx, idx_hbm.at[pl.ds(row0, K)])
                pltpu.sync_copy(xb, x_hbm.at[pl.ds(row0, K)])
                pltpu.sync_copy(yb, y_hbm.at[pl.ds(row0, K)])
                pltpu.sync_copy(zb, z_hbm.at[pl.ds(row0, K)])

                empty = jnp.where(cnt == 0, jnp.int32(1), jnp.int32(0))
                flags_vec = jnp.where(iota == bj, empty, flags_vec)

        for r in range(NCH):
            pltpu.make_async_copy(
                fbuf.at[r], feat_hbm.at[pl.ds(0, GCH)], osems[r]).wait()

        flags[...] = flags_vec
        pltpu.sync_copy(flags, flag_hbm.at[pl.ds(base_box, BPW)])

    return pool_kernel(pts_t, bparams, ftab)


def kernel(points, point_features, boxes3d):
    B, N, _ = points.shape
    M = boxes3d.shape[1]
    C = point_features.shape[2]
    K = _NUM_SAMPLED

    pts_t = jnp.transpose(points, (0, 2, 1)).reshape(B * 3, N)
    rz = boxes3d[..., 6]
    half = (boxes3d[..., 3:6] + 2.0 * _EXTRA) / 2.0
    zcol = jnp.zeros_like(rz)
    bparams = jnp.stack(
        [boxes3d[..., 0], boxes3d[..., 1], boxes3d[..., 2],
         half[..., 0], half[..., 1], half[..., 2],
         jnp.cos(-rz), jnp.sin(-rz)] + [zcol] * (_L - 8),
        axis=-1).reshape(B * M, _L)
    ftab = jnp.concatenate(
        [point_features, jnp.zeros((B, 8, C), jnp.float32)], axis=1
    ).reshape(B * (N + 8), C)

    feat, x, y, z, flags, idx = _sc_pool(
        pts_t, bparams, ftab, B=B, N=N, M=M, C=C)

    xyz = jnp.stack([x, y, z], axis=-1).reshape(B, M, K, 3)
    pooled = jnp.concatenate([xyz, feat.reshape(B, M, K, C)], axis=-1)
    return (pooled, flags.reshape(B, M), idx.reshape(B, M, K))


# speedup vs baseline: 1.1513x; 1.0208x over previous
import functools

import jax
import jax.numpy as jnp
from jax import lax
from jax.experimental import pallas as pl
from jax.experimental.pallas import tpu as pltpu
from jax.experimental.pallas import tpu_sc as plsc

_NUM_SAMPLED = 512
_EXTRA = 1.0
_L = 16
_GRP = 4


def _sc_pool(pts_t, bparams, ftab, *, B, N, M, C):
    NC, NS = 2, 16
    NW = NC * NS
    BOXES = B * M
    BPW = BOXES // NW
    K = _NUM_SAMPLED
    GCH = 128
    NCH = K // GCH

    mesh = plsc.VectorSubcoreMesh(
        core_axis_name="c", subcore_axis_name="s",
        num_cores=NC, num_subcores=NS)

    @functools.partial(
        pl.kernel,
        out_type=(
            jax.ShapeDtypeStruct((BOXES * K, C), jnp.float32),
            jax.ShapeDtypeStruct((BOXES * K,), jnp.float32),
            jax.ShapeDtypeStruct((BOXES * K,), jnp.float32),
            jax.ShapeDtypeStruct((BOXES * K,), jnp.float32),
            jax.ShapeDtypeStruct((BOXES,), jnp.int32),
            jax.ShapeDtypeStruct((BOXES * K,), jnp.int32),
        ),
        mesh=mesh,
        compiler_params=pltpu.CompilerParams(needs_layout_passes=False),
        scratch_types=[
            pltpu.VMEM((N,), jnp.float32),
            pltpu.VMEM((N,), jnp.float32),
            pltpu.VMEM((N,), jnp.float32),
            pltpu.VMEM((BPW, _L), jnp.float32),
            pltpu.VMEM((_GRP, K + _L), jnp.int32),
            pltpu.VMEM((NCH, GCH), jnp.int32),
            pltpu.VMEM((K,), jnp.int32),
            pltpu.VMEM((NCH, GCH, C), jnp.float32),
            pltpu.VMEM((K,), jnp.float32),
            pltpu.VMEM((K,), jnp.float32),
            pltpu.VMEM((K,), jnp.float32),
            pltpu.VMEM((BPW,), jnp.int32),
            pltpu.SemaphoreType.DMA,
            pltpu.SemaphoreType.DMA,
            pltpu.SemaphoreType.DMA,
            pltpu.SemaphoreType.DMA,
            pltpu.SemaphoreType.DMA,
            pltpu.SemaphoreType.DMA,
            pltpu.SemaphoreType.DMA,
            pltpu.SemaphoreType.DMA,
        ],
    )
    def pool_kernel(pts_hbm, bp_hbm, ftab_hbm,
                    feat_hbm, x_hbm, y_hbm, z_hbm, flag_hbm, idx_hbm,
                    xs, ys, zs, bp, bufs, gidx, oidx, fbuf,
                    xb, yb, zb, flags, gs0, gs1, gs2, gs3,
                    os0, os1, os2, os3):
        wid = lax.axis_index("s") * NC + lax.axis_index("c")
        base_box = wid * BPW
        batch = base_box // M
        pltpu.sync_copy(pts_hbm.at[batch * 3 + 0], xs)
        pltpu.sync_copy(pts_hbm.at[batch * 3 + 1], ys)
        pltpu.sync_copy(pts_hbm.at[batch * 3 + 2], zs)
        pltpu.sync_copy(bp_hbm.at[pl.ds(base_box, BPW)], bp)
        boff = batch * N
        iota = lax.iota(jnp.int32, _L)
        gsems = [gs0, gs1, gs2, gs3]
        osems = [os0, os1, os2, os3]
        flags_vec = jnp.zeros((_L,), jnp.int32)

        for g in range(BPW // _GRP):
            prm = []
            for t in range(_GRP):
                pv = bp[g * _GRP + t]
                prm.append((pv[0], pv[1], pv[2], pv[3], pv[4], pv[5],
                            pv[6], pv[7]))

            def step(i, cs, prm=prm):
                off = i * _L
                px = xs[pl.ds(off, _L)]
                py = ys[pl.ds(off, _L)]
                pz = zs[pl.ds(off, _L)]
                ivec = off + iota
                ncs = []
                for t, (cx, cy, cz, hx, hy, hz, ca, sa) in enumerate(prm):
                    cnt = cs[t]
                    sx = px - cx
                    sy = py - cy
                    lx = sx * ca - sy * sa
                    ly = sx * sa + sy * ca
                    m = ((jnp.abs(pz - cz) <= hz)
                         & (lx > -hx) & (lx < hx)
                         & (ly > -hy) & (ly < hy))
                    mi = jnp.where(m, jnp.int32(1), jnp.int32(0))
                    incl = plsc.cumsum(mi)
                    mm = m & lax.broadcast(cnt < K, (_L,))
                    plsc.store_scatter(
                        bufs, [lax.broadcast(jnp.int32(t), (_L,)),
                               cnt + incl - 1],
                        ivec, mask=mm)
                    ncs.append(cnt + incl[_L - 1])
                return tuple(ncs)

            cs = lax.fori_loop(0, N // _L, step,
                               tuple(jnp.int32(0) for _ in range(_GRP)))

            for t in range(_GRP):
                bj = g * _GRP + t
                cnt = cs[t]
                nonempty = cnt > 0
                safe = lax.broadcast(jnp.maximum(cnt, 1), (_L,))
                fzero = jnp.float32(0.0)
                tb = lax.broadcast(jnp.int32(t), (_L,))
                cpl = GCH // _L

                def samp(c, carry, tb=tb, safe=safe, nonempty=nonempty,
                         fzero=fzero):
                    kv = iota + c * _L
                    p = lax.rem(kv, safe)
                    gi = plsc.load_gather(bufs, [tb, p])
                    gsafe = jnp.where(nonempty, gi, 0)
                    oidx[pl.ds(c * _L, _L)] = gsafe
                    row = lax.broadcast(c // cpl, (_L,))
                    col = lax.rem(c, cpl) * _L + iota
                    plsc.store_scatter(gidx, [row, col], gsafe + boff)
                    xb[pl.ds(c * _L, _L)] = jnp.where(
                        nonempty, plsc.load_gather(xs, [gsafe]), fzero)
                    yb[pl.ds(c * _L, _L)] = jnp.where(
                        nonempty, plsc.load_gather(ys, [gsafe]), fzero)
                    zb[pl.ds(c * _L, _L)] = jnp.where(
                        nonempty, plsc.load_gather(zs, [gsafe]), fzero)
                    return carry

                lax.fori_loop(0, K // _L, samp, jnp.int32(0))

                row0 = (base_box + bj) * K
                gcps = []
                for r in range(NCH):
                    if bj > 0:
                        pltpu.make_async_copy(
                            fbuf.at[r],
                            feat_hbm.at[pl.ds(0, GCH)], osems[r]).wait()
                    gcps.append(pltpu.async_copy(
                        ftab_hbm.at[gidx.at[r]], fbuf.at[r], gsems[r]))
                for r in range(NCH):
                    gcps[r].wait()

                    @pl.when(jnp.logical_not(nonempty))
                    def _(r=r):
                        zvec = lax.broadcast(jnp.float32(0.0), (_L,))

                        def zfill(q, carry):
                            for v in range(C // _L):
                                fbuf[r, q, pl.ds(v * _L, _L)] = zvec
                            return carry

                        lax.fori_loop(0, GCH, zfill, jnp.int32(0))

                    pltpu.async_copy(
                        fbuf.at[r],
                        feat_hbm.at[pl.ds(row0 + r * GCH, GCH)], osems[r])
                pltpu.sync_copy(oidx, idx_hbm.at[pl.ds(row0, K)])
                pltpu.sync_copy(xb, x_hbm.at[pl.ds(row0, K)])
                pltpu.sync_copy(yb, y_hbm.at[pl.ds(row0, K)])
                pltpu.sync_copy(zb, z_hbm.at[pl.ds(row0, K)])

                empty = jnp.where(cnt == 0, jnp.int32(1), jnp.int32(0))
                flags_vec = jnp.where(iota == bj, empty, flags_vec)

        for r in range(NCH):
            pltpu.make_async_copy(
                fbuf.at[r], feat_hbm.at[pl.ds(0, GCH)], osems[r]).wait()

        flags[...] = flags_vec
        pltpu.sync_copy(flags, flag_hbm.at[pl.ds(base_box, BPW)])

    return pool_kernel(pts_t, bparams, ftab)


def kernel(points, point_features, boxes3d):
    B, N, _ = points.shape
    M = boxes3d.shape[1]
    C = point_features.shape[2]
    K = _NUM_SAMPLED

    pts_t = jnp.transpose(points, (0, 2, 1)).reshape(B * 3, N)
    rz = boxes3d[..., 6]
    half = (boxes3d[..., 3:6] + 2.0 * _EXTRA) / 2.0
    zcol = jnp.zeros_like(rz)
    bparams = jnp.stack(
        [boxes3d[..., 0], boxes3d[..., 1], boxes3d[..., 2],
         half[..., 0], half[..., 1], half[..., 2],
         jnp.cos(-rz), jnp.sin(-rz)] + [zcol] * (_L - 8),
        axis=-1).reshape(B * M, _L)
    ftab = point_features.reshape(B * N, C)

    feat, x, y, z, flags, idx = _sc_pool(
        pts_t, bparams, ftab, B=B, N=N, M=M, C=C)

    xyz = jnp.stack([x, y, z], axis=-1).reshape(B, M, K, 3)
    pooled = jnp.concatenate([xyz, feat.reshape(B, M, K, C)], axis=-1)
    return (pooled, flags.reshape(B, M), idx.reshape(B, M, K))


# speedup vs baseline: 1.1551x; 1.0033x over previous
import functools

import jax
import jax.numpy as jnp
from jax import lax
from jax.experimental import pallas as pl
from jax.experimental.pallas import tpu as pltpu
from jax.experimental.pallas import tpu_sc as plsc

_NUM_SAMPLED = 512
_EXTRA = 1.0
_L = 16
_GRP = 4


def _sc_pool(pts_t, bparams, ftab, *, B, N, M, C):
    NC, NS = 2, 16
    NW = NC * NS
    BOXES = B * M
    BPW = BOXES // NW
    K = _NUM_SAMPLED
    GCH = 128
    NCH = K // GCH

    mesh = plsc.VectorSubcoreMesh(
        core_axis_name="c", subcore_axis_name="s",
        num_cores=NC, num_subcores=NS)

    @functools.partial(
        pl.kernel,
        out_type=(
            jax.ShapeDtypeStruct((BOXES * K, C), jnp.float32),
            jax.ShapeDtypeStruct((BOXES * K,), jnp.float32),
            jax.ShapeDtypeStruct((BOXES * K,), jnp.float32),
            jax.ShapeDtypeStruct((BOXES * K,), jnp.float32),
            jax.ShapeDtypeStruct((BOXES,), jnp.int32),
            jax.ShapeDtypeStruct((BOXES * K,), jnp.int32),
        ),
        mesh=mesh,
        compiler_params=pltpu.CompilerParams(needs_layout_passes=False),
        scratch_types=[
            pltpu.VMEM((N,), jnp.float32),
            pltpu.VMEM((N,), jnp.float32),
            pltpu.VMEM((N,), jnp.float32),
            pltpu.VMEM((BPW, _L), jnp.float32),
            pltpu.VMEM((_GRP, K + _L), jnp.int32),
            pltpu.VMEM((NCH, GCH), jnp.int32),
            pltpu.VMEM((K,), jnp.int32),
            pltpu.VMEM((NCH, GCH, C), jnp.float32),
            pltpu.VMEM((K,), jnp.float32),
            pltpu.VMEM((K,), jnp.float32),
            pltpu.VMEM((K,), jnp.float32),
            pltpu.VMEM((BPW,), jnp.int32),
            pltpu.SemaphoreType.DMA,
            pltpu.SemaphoreType.DMA,
            pltpu.SemaphoreType.DMA,
            pltpu.SemaphoreType.DMA,
            pltpu.SemaphoreType.DMA,
            pltpu.SemaphoreType.DMA,
            pltpu.SemaphoreType.DMA,
            pltpu.SemaphoreType.DMA,
        ],
    )
    def pool_kernel(pts_hbm, bp_hbm, ftab_hbm,
                    feat_hbm, x_hbm, y_hbm, z_hbm, flag_hbm, idx_hbm,
                    xs, ys, zs, bp, bufs, gidx, oidx, fbuf,
                    xb, yb, zb, flags, gs0, gs1, gs2, gs3,
                    os0, os1, os2, os3):
        wid = lax.axis_index("s") * NC + lax.axis_index("c")
        base_box = wid * BPW
        batch = base_box // M
        pltpu.sync_copy(pts_hbm.at[batch * 3 + 0], xs)
        pltpu.sync_copy(pts_hbm.at[batch * 3 + 1], ys)
        pltpu.sync_copy(pts_hbm.at[batch * 3 + 2], zs)
        pltpu.sync_copy(bp_hbm.at[pl.ds(base_box, BPW)], bp)
        boff = batch * N
        iota = lax.iota(jnp.int32, _L)
        gsems = [gs0, gs1, gs2, gs3]
        osems = [os0, os1, os2, os3]
        flags_vec = jnp.zeros((_L,), jnp.int32)

        for g in range(BPW // _GRP):
            prm = []
            for t in range(_GRP):
                pv = bp[g * _GRP + t]
                prm.append((pv[0], pv[1], pv[2], pv[3], pv[4], pv[5],
                            pv[6], pv[7]))

            def step(i, cs, prm=prm):
                off = i * _L
                px = xs[pl.ds(off, _L)]
                py = ys[pl.ds(off, _L)]
                pz = zs[pl.ds(off, _L)]
                ivec = off + iota
                ncs = []
                for t, (cx, cy, cz, hx, hy, hz, ca, sa) in enumerate(prm):
                    cnt = cs[t]
                    sx = px - cx
                    sy = py - cy
                    lx = sx * ca - sy * sa
                    ly = sx * sa + sy * ca
                    m = ((jnp.abs(pz - cz) <= hz)
                         & (lx > -hx) & (lx < hx)
                         & (ly > -hy) & (ly < hy))
                    mi = jnp.where(m, jnp.int32(1), jnp.int32(0))
                    incl = plsc.cumsum(mi)
                    mm = m & lax.broadcast(cnt < K, (_L,))
                    plsc.store_scatter(
                        bufs, [lax.broadcast(jnp.int32(t), (_L,)),
                               cnt + incl - 1],
                        ivec, mask=mm)
                    ncs.append(cnt + incl[_L - 1])
                return tuple(ncs)

            cs = lax.fori_loop(0, N // _L, step,
                               tuple(jnp.int32(0) for _ in range(_GRP)))

            for t in range(_GRP):
                bj = g * _GRP + t
                cnt = cs[t]
                nonempty = cnt > 0
                safe = lax.broadcast(jnp.maximum(cnt, 1), (_L,))
                fzero = jnp.float32(0.0)
                tb = lax.broadcast(jnp.int32(t), (_L,))
                cpl = GCH // _L

                @pl.when(cnt >= K)
                def _(t=t, tb=tb):
                    def samp(c, carry):
                        gi = bufs[t, pl.ds(c * _L, _L)]
                        oidx[pl.ds(c * _L, _L)] = gi
                        row = lax.broadcast(c // cpl, (_L,))
                        col = lax.rem(c, cpl) * _L + iota
                        plsc.store_scatter(gidx, [row, col], gi + boff)
                        xb[pl.ds(c * _L, _L)] = plsc.load_gather(xs, [gi])
                        yb[pl.ds(c * _L, _L)] = plsc.load_gather(ys, [gi])
                        zb[pl.ds(c * _L, _L)] = plsc.load_gather(zs, [gi])
                        return carry

                    lax.fori_loop(0, K // _L, samp, jnp.int32(0))

                @pl.when(cnt < K)
                def _(tb=tb, safe=safe, nonempty=nonempty, fzero=fzero):
                    def samp(c, carry):
                        kv = iota + c * _L
                        p = lax.rem(kv, safe)
                        gi = plsc.load_gather(bufs, [tb, p])
                        gsafe = jnp.where(nonempty, gi, 0)
                        oidx[pl.ds(c * _L, _L)] = gsafe
                        row = lax.broadcast(c // cpl, (_L,))
                        col = lax.rem(c, cpl) * _L + iota
                        plsc.store_scatter(gidx, [row, col], gsafe + boff)
                        xb[pl.ds(c * _L, _L)] = jnp.where(
                            nonempty, plsc.load_gather(xs, [gsafe]), fzero)
                        yb[pl.ds(c * _L, _L)] = jnp.where(
                            nonempty, plsc.load_gather(ys, [gsafe]), fzero)
                        zb[pl.ds(c * _L, _L)] = jnp.where(
                            nonempty, plsc.load_gather(zs, [gsafe]), fzero)
                        return carry

                    lax.fori_loop(0, K // _L, samp, jnp.int32(0))

                row0 = (base_box + bj) * K
                gcps = []
                for r in range(NCH):
                    if bj > 0:
                        pltpu.make_async_copy(
                            fbuf.at[r],
                            feat_hbm.at[pl.ds(0, GCH)], osems[r]).wait()
                    gcps.append(pltpu.async_copy(
                        ftab_hbm.at[gidx.at[r]], fbuf.at[r], gsems[r]))
                for r in range(NCH):
                    gcps[r].wait()

                    @pl.when(jnp.logical_not(nonempty))
                    def _(r=r):
                        zvec = lax.broadcast(jnp.float32(0.0), (_L,))

                        def zfill(q, carry):
                            for v in range(C // _L):
                                fbuf[r, q, pl.ds(v * _L, _L)] = zvec
                            return carry

                        lax.fori_loop(0, GCH, zfill, jnp.int32(0))

                    pltpu.async_copy(
                        fbuf.at[r],
                        feat_hbm.at[pl.ds(row0 + r * GCH, GCH)], osems[r])
                pltpu.sync_copy(oidx, idx_hbm.at[pl.ds(row0, K)])
                pltpu.sync_copy(xb, x_hbm.at[pl.ds(row0, K)])
                pltpu.sync_copy(yb, y_hbm.at[pl.ds(row0, K)])
                pltpu.sync_copy(zb, z_hbm.at[pl.ds(row0, K)])

                empty = jnp.where(cnt == 0, jnp.int32(1), jnp.int32(0))
                flags_vec = jnp.where(iota == bj, empty, flags_vec)

        for r in range(NCH):
            pltpu.make_async_copy(
                fbuf.at[r], feat_hbm.at[pl.ds(0, GCH)], osems[r]).wait()

        flags[...] = flags_vec
        pltpu.sync_copy(flags, flag_hbm.at[pl.ds(base_box, BPW)])

    return pool_kernel(pts_t, bparams, ftab)


def kernel(points, point_features, boxes3d):
    B, N, _ = points.shape
    M = boxes3d.shape[1]
    C = point_features.shape[2]
    K = _NUM_SAMPLED

    pts_t = jnp.transpose(points, (0, 2, 1)).reshape(B * 3, N)
    rz = boxes3d[..., 6]
    half = (boxes3d[..., 3:6] + 2.0 * _EXTRA) / 2.0
    zcol = jnp.zeros_like(rz)
    bparams = jnp.stack(
        [boxes3d[..., 0], boxes3d[..., 1], boxes3d[..., 2],
         half[..., 0], half[..., 1], half[..., 2],
         jnp.cos(-rz), jnp.sin(-rz)] + [zcol] * (_L - 8),
        axis=-1).reshape(B * M, _L)
    ftab = point_features.reshape(B * N, C)

    feat, x, y, z, flags, idx = _sc_pool(
        pts_t, bparams, ftab, B=B, N=N, M=M, C=C)

    xyz = jnp.stack([x, y, z], axis=-1).reshape(B, M, K, 3)
    pooled = jnp.concatenate([xyz, feat.reshape(B, M, K, C)], axis=-1)
    return (pooled, flags.reshape(B, M), idx.reshape(B, M, K))
